# Initial kernel scaffold; baseline (speedup 1.0000x reference)
import jax
import jax.numpy as jnp
from jax.experimental import pallas as pl


def kernel(x, edge_index, edge_attr, params):
    raise NotImplementedError("write your pallas kernel here")


# speedup vs baseline: 9.6707x; 9.6707x over previous
import functools

import jax
import jax.numpy as jnp
from jax import lax
from jax.experimental import pallas as pl
from jax.experimental.pallas import tpu as pltpu
from jax.experimental.pallas import tpu_sc as plsc

N = 10000
NP = 10240
D = 128
HD = 64
E = 320000
NSUB = 16
NCORE = 2
CH = 128
NCH = 157
EP = NSUB * NCH * CH
NB = 1024
GRID = NP // NB
STRIPE = NP // NSUB

_SC_MESH = plsc.VectorSubcoreMesh(core_axis_name="c", subcore_axis_name="s")
_SC_PARAMS = pltpu.CompilerParams(use_tc_tiling_on_sc=False)


@functools.partial(
    pl.kernel,
    out_type=jax.ShapeDtypeStruct((NCORE, 4, NP, 16), jnp.float32),
    mesh=_SC_MESH,
    compiler_params=_SC_PARAMS,
    scratch_types=[
        pltpu.VMEM((NCH, CH), jnp.int32),
        pltpu.VMEM((NCH, CH), jnp.float32),
        pltpu.VMEM((CH, 16), jnp.float32),
        pltpu.VMEM((CH, 16), jnp.float32),
        pltpu.VMEM_SHARED((NP, 16), jnp.float32),
    ],
)
def _deg_kernel(col_hbm, w_hbm, out_hbm, col_v, w_v, wrow_v, zbuf, deg_sp):
    c = lax.axis_index("c")
    s = lax.axis_index("s")

    def zrow(i, _):
        zbuf[i, pl.ds(0, 16)] = jnp.zeros((16,), jnp.float32)
        return 0
    lax.fori_loop(0, CH, zrow, 0)
    for a in range(4):
        for t in range(STRIPE // CH):
            pltpu.sync_copy(
                zbuf, deg_sp.at[pl.ds(s * STRIPE + t * CH, CH), :])
        plsc.subcore_barrier()
        pltpu.sync_copy(col_hbm.at[a, s], col_v)
        pltpu.sync_copy(w_hbm.at[a, s], w_v)

        def chunk(t, _):
            j = 2 * t + c

            @pl.when(j < NCH)
            def _():
                def group(g, _):
                    wv = w_v[j, pl.ds(g * 16, 16)]
                    for k in range(16):
                        wrow_v[g * 16 + k, pl.ds(0, 16)] = jnp.full(
                            (16,), wv[k], jnp.float32)
                    return 0
                lax.fori_loop(0, CH // 16, group, 0)
                pltpu.sync_copy(wrow_v, deg_sp.at[col_v.at[j]], add=True)
            return 0
        lax.fori_loop(0, (NCH + 1) // 2, chunk, 0)
        plsc.subcore_barrier()
        pltpu.sync_copy(deg_sp.at[pl.ds(s * STRIPE, STRIPE), :],
                        out_hbm.at[c, a, pl.ds(s * STRIPE, STRIPE), :])
        plsc.subcore_barrier()


@functools.partial(
    pl.kernel,
    out_type=jax.ShapeDtypeStruct((NCORE, NP, HD), jnp.float32),
    mesh=_SC_MESH,
    compiler_params=_SC_PARAMS,
    scratch_types=[
        pltpu.VMEM((NCH, CH), jnp.int32),
        pltpu.VMEM((NCH, CH), jnp.int32),
        pltpu.VMEM((NCH, CH), jnp.float32),
        pltpu.VMEM((CH, HD), jnp.float32),
        pltpu.VMEM((CH, HD), jnp.float32),
        pltpu.VMEM_SHARED((NP, HD), jnp.float32),
        pltpu.SemaphoreType.DMA,
    ],
)
def _conv_kernel(hwd_hbm, row_hbm, col_hbm, w_hbm, out_hbm,
                 row_v, col_v, w_v, rows_v, zbuf, agg_sp, sem):
    c = lax.axis_index("c")
    s = lax.axis_index("s")
    pltpu.sync_copy(row_hbm.at[s], row_v)
    pltpu.sync_copy(col_hbm.at[s], col_v)
    pltpu.sync_copy(w_hbm.at[s], w_v)
    def zrow(i, _):
        for f in range(HD // 16):
            zbuf[i, pl.ds(f * 16, 16)] = jnp.zeros((16,), jnp.float32)
        return 0
    lax.fori_loop(0, CH, zrow, 0)
    for t in range(STRIPE // CH):
        pltpu.sync_copy(zbuf, agg_sp.at[pl.ds(s * STRIPE + t * CH, CH), :])
    plsc.subcore_barrier()

    def chunk(j, _):
        pltpu.async_copy(hwd_hbm.at[c].at[row_v.at[j]], rows_v, sem).wait()

        def group(g, _):
            wv = w_v[j, pl.ds(g * 16, 16)]
            for k in range(16):
                w = wv[k]
                e = g * 16 + k
                for f in range(HD // 16):
                    rows_v[e, pl.ds(f * 16, 16)] = (
                        rows_v[e, pl.ds(f * 16, 16)] * w)
            return 0
        lax.fori_loop(0, CH // 16, group, 0)
        pltpu.sync_copy(rows_v, agg_sp.at[col_v.at[j]], add=True)
        return 0
    lax.fori_loop(0, NCH, chunk, 0)
    plsc.subcore_barrier()
    pltpu.sync_copy(agg_sp.at[pl.ds(s * STRIPE, STRIPE), :],
                    out_hbm.at[c, pl.ds(s * STRIPE, STRIPE), :])


def _row_spec(shape):
    nd = len(shape)
    blk = (NB,) + tuple(shape[1:])
    return pl.BlockSpec(blk, lambda i: (i,) + (0,) * (nd - 1))


def _full_spec(shape):
    nd = len(shape)
    return pl.BlockSpec(tuple(shape), lambda i: (0,) * nd)


def _split_spec():
    return pl.BlockSpec((NCORE, NB, HD), lambda i: (0, i, 0))


def _dinv_spec():
    return pl.BlockSpec((4, NB), lambda i: (0, i))


def _k1_body(x_ref, linw_ref, linb_ref, convw_ref, degp_ref, comb_ref,
             acc_ref, hwd_ref, dinv_ref):
    h = jnp.maximum(
        jnp.dot(x_ref[...], linw_ref[...],
                preferred_element_type=jnp.float32) + linb_ref[...], 0.0)
    deg = degp_ref[0, :, :, 0] + degp_ref[1, :, :, 0]
    dinv = jnp.where(deg > 0, lax.rsqrt(deg), 0.0)
    dinv_ref[...] = dinv
    acc_ref[...] = comb_ref[0, 0] * h
    hw = jnp.dot(h, convw_ref[...], preferred_element_type=jnp.float32)
    hwd = hw * dinv[0][:, None]
    hwd_ref[0] = hwd[:, :HD]
    hwd_ref[1] = hwd[:, HD:]


def _tc_k1(x_p, lin_w, lin_b, conv_w0, degp, comb):
    return pl.pallas_call(
        _k1_body,
        grid=(GRID,),
        in_specs=[
            _row_spec((NP, D)),
            _full_spec((D, D)),
            _full_spec((1, D)),
            _full_spec((D, D)),
            pl.BlockSpec((NCORE, 4, NB, 16), lambda i: (0, 0, i, 0)),
            _full_spec((1, 5)),
        ],
        out_specs=[_row_spec((NP, D)), _split_spec(), _dinv_spec()],
        out_shape=[
            jax.ShapeDtypeStruct((NP, D), jnp.float32),
            jax.ShapeDtypeStruct((NCORE, NP, HD), jnp.float32),
            jax.ShapeDtypeStruct((4, NP), jnp.float32),
        ],
    )(x_p, lin_w, lin_b, conv_w0, degp, comb)


def _step_body(a, agg_ref, dinv_ref, convb_ref, convw_ref, accin_ref,
               comb_ref, acc_ref, hwd_ref):
    aggf = jnp.concatenate([agg_ref[0], agg_ref[1]], axis=1)
    h = jnp.maximum(
        dinv_ref[a - 1][:, None] * aggf + convb_ref[...], 0.0)
    acc_ref[...] = accin_ref[...] + comb_ref[0, a] * h
    hw = jnp.dot(h, convw_ref[...], preferred_element_type=jnp.float32)
    hwd = hw * dinv_ref[a][:, None]
    hwd_ref[0] = hwd[:, :HD]
    hwd_ref[1] = hwd[:, HD:]


def _tc_step(a, agg, dinv, conv_b_prev, conv_w, acc, comb):
    return pl.pallas_call(
        functools.partial(_step_body, a),
        grid=(GRID,),
        in_specs=[
            _split_spec(),
            _dinv_spec(),
            _full_spec((1, D)),
            _full_spec((D, D)),
            _row_spec((NP, D)),
            _full_spec((1, 5)),
        ],
        out_specs=[_row_spec((NP, D)), _split_spec()],
        out_shape=[
            jax.ShapeDtypeStruct((NP, D), jnp.float32),
            jax.ShapeDtypeStruct((NCORE, NP, HD), jnp.float32),
        ],
    )(agg, dinv, conv_b_prev, conv_w, acc, comb)


def _final_body(agg_ref, dinv_ref, convb_ref, accin_ref, comb_ref, out_ref):
    aggf = jnp.concatenate([agg_ref[0], agg_ref[1]], axis=1)
    h = jnp.maximum(dinv_ref[3][:, None] * aggf + convb_ref[...], 0.0)
    out = accin_ref[...] + comb_ref[0, 4] * h
    m = jnp.max(out, axis=-1, keepdims=True)
    lse = jnp.log(jnp.sum(jnp.exp(out - m), axis=-1, keepdims=True)) + m
    out_ref[...] = out - lse


def _tc_final(agg, dinv, conv_b3, acc, comb):
    return pl.pallas_call(
        _final_body,
        grid=(GRID,),
        in_specs=[
            _split_spec(),
            _dinv_spec(),
            _full_spec((1, D)),
            _row_spec((NP, D)),
            _full_spec((1, 5)),
        ],
        out_specs=_row_spec((NP, D)),
        out_shape=jax.ShapeDtypeStruct((NP, D), jnp.float32),
    )(agg, dinv, conv_b3, acc, comb)


def kernel(x, edge_index, edge_attr, params):
    lin_w = params["lin_W"][-1]
    lin_b = params["lin_b"][-1].reshape(1, D)
    conv_w = params["conv_W"][-1]
    conv_b = params["conv_b"][-1]
    comb = params["comb_w"][-1].reshape(1, 5)

    row = edge_index[:, 0, :].astype(jnp.int32)
    col = edge_index[:, 1, :].astype(jnp.int32)
    pad = EP - E
    pad_idx = (jnp.arange(pad, dtype=jnp.int32) * 7) % N
    pad_idx4 = jnp.broadcast_to(pad_idx, (4, pad))
    shp = (4, NSUB, NCH, CH)
    row_t = jnp.concatenate([row, pad_idx4], axis=1).reshape(shp)
    col_t = jnp.concatenate([col, pad_idx4], axis=1).reshape(shp)
    w_t = jnp.concatenate(
        [edge_attr.astype(jnp.float32), jnp.zeros((4, pad), jnp.float32)],
        axis=1).reshape(shp)

    x_p = jnp.concatenate([x, jnp.zeros((NP - N, D), jnp.float32)], axis=0)

    degp = _deg_kernel(col_t, w_t)
    acc, hwd, dinv = _tc_k1(x_p, lin_w, lin_b, conv_w[0], degp, comb)

    for a in range(1, 4):
        agg = _conv_kernel(hwd, row_t[a - 1], col_t[a - 1], w_t[a - 1])
        acc, hwd = _tc_step(a, agg, dinv, conv_b[a - 1].reshape(1, D),
                            conv_w[a], acc, comb)
    agg = _conv_kernel(hwd, row_t[3], col_t[3], w_t[3])
    out = _tc_final(agg, dinv, conv_b[3].reshape(1, D), acc, comb)
    return out[:N]


# speedup vs baseline: 12.5833x; 1.3012x over previous
import functools

import jax
import jax.numpy as jnp
from jax import lax
from jax.experimental import pallas as pl
from jax.experimental.pallas import tpu as pltpu
from jax.experimental.pallas import tpu_sc as plsc

N = 10000
NP = 10240
D = 128
HD = 64
E = 320000
NSUB = 16
NCORE = 2
CH = 128
NCH = 158
EP = NSUB * NCH * CH
NB = 1024
GRID = NP // NB
STRIPE = NP // NSUB

_SC_MESH = plsc.VectorSubcoreMesh(core_axis_name="c", subcore_axis_name="s")
_SC_PARAMS = pltpu.CompilerParams(use_tc_tiling_on_sc=False)


@functools.partial(
    pl.kernel,
    out_type=jax.ShapeDtypeStruct((NCORE, 4, NP, 16), jnp.float32),
    mesh=_SC_MESH,
    compiler_params=_SC_PARAMS,
    scratch_types=[
        pltpu.VMEM((NCH, CH), jnp.int32),
        pltpu.VMEM((NCH, CH), jnp.float32),
        pltpu.VMEM((CH, 16), jnp.float32),
        pltpu.VMEM((CH, 16), jnp.float32),
        pltpu.VMEM_SHARED((NP, 16), jnp.float32),
    ],
)
def _deg_kernel(col_hbm, w_hbm, out_hbm, col_v, w_v, wrow_v, zbuf, deg_sp):
    c = lax.axis_index("c")
    s = lax.axis_index("s")

    def zrow(i, _):
        zbuf[i, pl.ds(0, 16)] = jnp.zeros((16,), jnp.float32)
        return 0
    lax.fori_loop(0, CH, zrow, 0)
    for a in range(4):
        for t in range(STRIPE // CH):
            pltpu.sync_copy(
                zbuf, deg_sp.at[pl.ds(s * STRIPE + t * CH, CH), :])
        plsc.subcore_barrier()
        pltpu.sync_copy(col_hbm.at[a, s], col_v)
        pltpu.sync_copy(w_hbm.at[a, s], w_v)

        def chunk(t, _):
            j = 2 * t + c

            @pl.when(j < NCH)
            def _():
                def group(g, _):
                    wv = w_v[j, pl.ds(g * 16, 16)]
                    for k in range(16):
                        wrow_v[g * 16 + k, pl.ds(0, 16)] = jnp.full(
                            (16,), wv[k], jnp.float32)
                    return 0
                lax.fori_loop(0, CH // 16, group, 0)
                pltpu.sync_copy(wrow_v, deg_sp.at[col_v.at[j]], add=True)
            return 0
        lax.fori_loop(0, (NCH + 1) // 2, chunk, 0)
        plsc.subcore_barrier()
        pltpu.sync_copy(deg_sp.at[pl.ds(s * STRIPE, STRIPE), :],
                        out_hbm.at[c, a, pl.ds(s * STRIPE, STRIPE), :])
        plsc.subcore_barrier()


@functools.partial(
    pl.kernel,
    out_type=jax.ShapeDtypeStruct((NCORE, NP, HD), jnp.float32),
    mesh=_SC_MESH,
    compiler_params=_SC_PARAMS,
    scratch_types=[
        pltpu.VMEM((NCH, CH), jnp.int32),
        pltpu.VMEM((NCH, CH), jnp.int32),
        pltpu.VMEM((NCH, CH), jnp.float32),
        pltpu.VMEM((CH, HD), jnp.float32),
        pltpu.VMEM((CH, HD), jnp.float32),
        pltpu.VMEM((CH, HD), jnp.float32),
        pltpu.VMEM_SHARED((NP, HD), jnp.float32),
        pltpu.SemaphoreType.DMA,
        pltpu.SemaphoreType.DMA,
        pltpu.SemaphoreType.DMA,
        pltpu.SemaphoreType.DMA,
    ],
)
def _conv_kernel(hwd_hbm, row_hbm, col_hbm, w_hbm, out_hbm,
                 row_v, col_v, w_v, rows_a, rows_b, zbuf, agg_sp,
                 gsa, gsb, ssa, ssb):
    c = lax.axis_index("c")
    s = lax.axis_index("s")
    pltpu.sync_copy(row_hbm.at[s], row_v)
    pltpu.sync_copy(col_hbm.at[s], col_v)
    pltpu.sync_copy(w_hbm.at[s], w_v)
    def zrow(i, _):
        for f in range(HD // 16):
            zbuf[i, pl.ds(f * 16, 16)] = jnp.zeros((16,), jnp.float32)
        return 0
    lax.fori_loop(0, CH, zrow, 0)
    for t in range(STRIPE // CH):
        pltpu.sync_copy(zbuf, agg_sp.at[pl.ds(s * STRIPE + t * CH, CH), :])
    plsc.subcore_barrier()

    def gather(j, buf, sem):
        pltpu.async_copy(hwd_hbm.at[c].at[row_v.at[j]], buf, sem)

    def gather_wait(buf, sem):
        pltpu.make_async_copy(hwd_hbm.at[c].at[row_v.at[0]], buf, sem).wait()

    def scatter(j, buf, sem):
        pltpu.async_copy(buf, agg_sp.at[col_v.at[j]], sem, add=True)

    def scatter_wait(buf, sem):
        pltpu.make_async_copy(buf, agg_sp.at[col_v.at[0]], sem).wait()

    def scale(j, buf):
        def group(g, _):
            wv = w_v[j, pl.ds(g * 16, 16)]
            for k in range(16):
                w = wv[k]
                e = g * 16 + k
                for f in range(HD // 16):
                    buf[e, pl.ds(f * 16, 16)] = buf[e, pl.ds(f * 16, 16)] * w
            return 0
        lax.fori_loop(0, CH // 16, group, 0)

    gather(0, rows_a, gsa)
    gather_wait(rows_a, gsa)
    gather(1, rows_b, gsb)
    scale(0, rows_a)
    scatter(0, rows_a, ssa)
    gather_wait(rows_b, gsb)
    scatter_wait(rows_a, ssa)
    gather(2, rows_a, gsa)
    scale(1, rows_b)
    scatter(1, rows_b, ssb)

    def pipe(t, _):
        j0 = 2 * t
        j1 = j0 + 1
        j2 = j0 + 2
        gather_wait(rows_a, gsa)
        scatter_wait(rows_b, ssb)
        gather(j1, rows_b, gsb)
        scale(j0, rows_a)
        scatter(j0, rows_a, ssa)
        gather_wait(rows_b, gsb)
        scatter_wait(rows_a, ssa)

        @pl.when(j2 < NCH)
        def _():
            gather(j2, rows_a, gsa)
        scale(j1, rows_b)
        scatter(j1, rows_b, ssb)
        return 0
    lax.fori_loop(1, NCH // 2, pipe, 0)
    scatter_wait(rows_b, ssb)
    plsc.subcore_barrier()
    pltpu.sync_copy(agg_sp.at[pl.ds(s * STRIPE, STRIPE), :],
                    out_hbm.at[c, pl.ds(s * STRIPE, STRIPE), :])


def _row_spec(shape):
    nd = len(shape)
    blk = (NB,) + tuple(shape[1:])
    return pl.BlockSpec(blk, lambda i: (i,) + (0,) * (nd - 1))


def _full_spec(shape):
    nd = len(shape)
    return pl.BlockSpec(tuple(shape), lambda i: (0,) * nd)


def _split_spec():
    return pl.BlockSpec((NCORE, NB, HD), lambda i: (0, i, 0))


def _dinv_spec():
    return pl.BlockSpec((4, NB), lambda i: (0, i))


def _k1_body(x_ref, linw_ref, linb_ref, convw_ref, degp_ref, comb_ref,
             acc_ref, hwd_ref, dinv_ref):
    h = jnp.maximum(
        jnp.dot(x_ref[...], linw_ref[...],
                preferred_element_type=jnp.float32) + linb_ref[...], 0.0)
    deg = degp_ref[0, :, :, 0] + degp_ref[1, :, :, 0]
    dinv = jnp.where(deg > 0, lax.rsqrt(deg), 0.0)
    dinv_ref[...] = dinv
    acc_ref[...] = comb_ref[0, 0] * h
    hw = jnp.dot(h, convw_ref[...], preferred_element_type=jnp.float32)
    hwd = hw * dinv[0][:, None]
    hwd_ref[0] = hwd[:, :HD]
    hwd_ref[1] = hwd[:, HD:]


def _tc_k1(x_p, lin_w, lin_b, conv_w0, degp, comb):
    return pl.pallas_call(
        _k1_body,
        grid=(GRID,),
        in_specs=[
            _row_spec((NP, D)),
            _full_spec((D, D)),
            _full_spec((1, D)),
            _full_spec((D, D)),
            pl.BlockSpec((NCORE, 4, NB, 16), lambda i: (0, 0, i, 0)),
            _full_spec((1, 5)),
        ],
        out_specs=[_row_spec((NP, D)), _split_spec(), _dinv_spec()],
        out_shape=[
            jax.ShapeDtypeStruct((NP, D), jnp.float32),
            jax.ShapeDtypeStruct((NCORE, NP, HD), jnp.float32),
            jax.ShapeDtypeStruct((4, NP), jnp.float32),
        ],
    )(x_p, lin_w, lin_b, conv_w0, degp, comb)


def _step_body(a, agg_ref, dinv_ref, convb_ref, convw_ref, accin_ref,
               comb_ref, acc_ref, hwd_ref):
    aggf = jnp.concatenate([agg_ref[0], agg_ref[1]], axis=1)
    h = jnp.maximum(
        dinv_ref[a - 1][:, None] * aggf + convb_ref[...], 0.0)
    acc_ref[...] = accin_ref[...] + comb_ref[0, a] * h
    hw = jnp.dot(h, convw_ref[...], preferred_element_type=jnp.float32)
    hwd = hw * dinv_ref[a][:, None]
    hwd_ref[0] = hwd[:, :HD]
    hwd_ref[1] = hwd[:, HD:]


def _tc_step(a, agg, dinv, conv_b_prev, conv_w, acc, comb):
    return pl.pallas_call(
        functools.partial(_step_body, a),
        grid=(GRID,),
        in_specs=[
            _split_spec(),
            _dinv_spec(),
            _full_spec((1, D)),
            _full_spec((D, D)),
            _row_spec((NP, D)),
            _full_spec((1, 5)),
        ],
        out_specs=[_row_spec((NP, D)), _split_spec()],
        out_shape=[
            jax.ShapeDtypeStruct((NP, D), jnp.float32),
            jax.ShapeDtypeStruct((NCORE, NP, HD), jnp.float32),
        ],
    )(agg, dinv, conv_b_prev, conv_w, acc, comb)


def _final_body(agg_ref, dinv_ref, convb_ref, accin_ref, comb_ref, out_ref):
    aggf = jnp.concatenate([agg_ref[0], agg_ref[1]], axis=1)
    h = jnp.maximum(dinv_ref[3][:, None] * aggf + convb_ref[...], 0.0)
    out = accin_ref[...] + comb_ref[0, 4] * h
    m = jnp.max(out, axis=-1, keepdims=True)
    lse = jnp.log(jnp.sum(jnp.exp(out - m), axis=-1, keepdims=True)) + m
    out_ref[...] = out - lse


def _tc_final(agg, dinv, conv_b3, acc, comb):
    return pl.pallas_call(
        _final_body,
        grid=(GRID,),
        in_specs=[
            _split_spec(),
            _dinv_spec(),
            _full_spec((1, D)),
            _row_spec((NP, D)),
            _full_spec((1, 5)),
        ],
        out_specs=_row_spec((NP, D)),
        out_shape=jax.ShapeDtypeStruct((NP, D), jnp.float32),
    )(agg, dinv, conv_b3, acc, comb)


def kernel(x, edge_index, edge_attr, params):
    lin_w = params["lin_W"][-1]
    lin_b = params["lin_b"][-1].reshape(1, D)
    conv_w = params["conv_W"][-1]
    conv_b = params["conv_b"][-1]
    comb = params["comb_w"][-1].reshape(1, 5)

    row = edge_index[:, 0, :].astype(jnp.int32)
    col = edge_index[:, 1, :].astype(jnp.int32)
    pad = EP - E
    pad_idx = (jnp.arange(pad, dtype=jnp.int32) * 7) % N
    pad_idx4 = jnp.broadcast_to(pad_idx, (4, pad))
    shp = (4, NSUB, NCH, CH)
    row_t = jnp.concatenate([row, pad_idx4], axis=1).reshape(shp)
    col_t = jnp.concatenate([col, pad_idx4], axis=1).reshape(shp)
    w_t = jnp.concatenate(
        [edge_attr.astype(jnp.float32), jnp.zeros((4, pad), jnp.float32)],
        axis=1).reshape(shp)

    x_p = jnp.concatenate([x, jnp.zeros((NP - N, D), jnp.float32)], axis=0)

    degp = _deg_kernel(col_t, w_t)
    acc, hwd, dinv = _tc_k1(x_p, lin_w, lin_b, conv_w[0], degp, comb)

    for a in range(1, 4):
        agg = _conv_kernel(hwd, row_t[a - 1], col_t[a - 1], w_t[a - 1])
        acc, hwd = _tc_step(a, agg, dinv, conv_b[a - 1].reshape(1, D),
                            conv_w[a], acc, comb)
    agg = _conv_kernel(hwd, row_t[3], col_t[3], w_t[3])
    out = _tc_final(agg, dinv, conv_b[3].reshape(1, D), acc, comb)
    return out[:N]


# speedup vs baseline: 28.1040x; 2.2334x over previous
import functools

import jax
import jax.numpy as jnp
from jax import lax
from jax.experimental import pallas as pl
from jax.experimental.pallas import tpu as pltpu
from jax.experimental.pallas import tpu_sc as plsc

N = 10000
NP = 10240
D = 128
HD = 64
E = 320000
NSUB = 16
NCORE = 2
CH = 128
NCH = 160
NH = NCH // 2
EP = NSUB * NCH * CH
NB = 1024
GRID = NP // NB
STRIPE = NP // NSUB

_SC_MESH = plsc.VectorSubcoreMesh(core_axis_name="c", subcore_axis_name="s")
_SC_PARAMS = pltpu.CompilerParams(use_tc_tiling_on_sc=False)


@functools.partial(
    pl.kernel,
    out_type=jax.ShapeDtypeStruct((NCORE, 4, NP, 16), jnp.float32),
    mesh=_SC_MESH,
    compiler_params=_SC_PARAMS,
    scratch_types=[
        pltpu.VMEM((NCH, CH), jnp.int32),
        pltpu.VMEM((NCH, CH), jnp.float32),
        pltpu.VMEM((CH, 16), jnp.float32),
        pltpu.VMEM((CH, 16), jnp.float32),
        pltpu.VMEM_SHARED((NP, 16), jnp.float32),
    ],
)
def _deg_kernel(col_hbm, w_hbm, out_hbm, col_v, w_v, wrow_v, zbuf, deg_sp):
    c = lax.axis_index("c")
    s = lax.axis_index("s")

    def zrow(i, _):
        zbuf[i, pl.ds(0, 16)] = jnp.zeros((16,), jnp.float32)
        return 0
    lax.fori_loop(0, CH, zrow, 0)
    for a in range(4):
        for t in range(STRIPE // CH):
            pltpu.sync_copy(
                zbuf, deg_sp.at[pl.ds(s * STRIPE + t * CH, CH), :])
        plsc.subcore_barrier()
        pltpu.sync_copy(col_hbm.at[a, s], col_v)
        pltpu.sync_copy(w_hbm.at[a, s], w_v)

        def chunk(t, _):
            j = 2 * t + c

            @pl.when(j < NCH)
            def _():
                def group(g, _):
                    wv = w_v[j, pl.ds(g * 16, 16)]
                    for k in range(16):
                        wrow_v[g * 16 + k, pl.ds(0, 16)] = jnp.full(
                            (16,), wv[k], jnp.float32)
                    return 0
                lax.fori_loop(0, CH // 16, group, 0)
                pltpu.sync_copy(wrow_v, deg_sp.at[col_v.at[j]], add=True)
            return 0
        lax.fori_loop(0, (NCH + 1) // 2, chunk, 0)
        plsc.subcore_barrier()
        pltpu.sync_copy(deg_sp.at[pl.ds(s * STRIPE, STRIPE), :],
                        out_hbm.at[c, a, pl.ds(s * STRIPE, STRIPE), :])
        plsc.subcore_barrier()


@functools.partial(
    pl.kernel,
    out_type=jax.ShapeDtypeStruct((NCORE, NP, HD), jnp.float32),
    mesh=_SC_MESH,
    compiler_params=_SC_PARAMS,
    scratch_types=[
        pltpu.VMEM((NH, CH), jnp.int32),
        pltpu.VMEM((NH, CH), jnp.int32),
        pltpu.VMEM((NH, CH), jnp.float32),
        pltpu.VMEM((CH, HD), jnp.float32),
        pltpu.VMEM((CH, HD), jnp.float32),
        pltpu.VMEM((CH, HD), jnp.float32),
        pltpu.VMEM((CH, HD), jnp.float32),
        pltpu.VMEM_SHARED((NP, HD), jnp.float32),
        pltpu.SemaphoreType.DMA,
        pltpu.SemaphoreType.DMA,
        pltpu.SemaphoreType.DMA,
        pltpu.SemaphoreType.DMA,
    ],
)
def _conv_kernel(hwd_hbm, row_hbm, col_hbm, w_hbm, out_hbm,
                 row_v, col_v, w_v, ga_v, gb_v, sa_v, sb_v, agg_sp,
                 gsa, gsb, ssa, ssb):
    c = lax.axis_index("c")
    s = lax.axis_index("s")
    def zrow(i, _):
        for f in range(HD // 16):
            sa_v[i, pl.ds(f * 16, 16)] = jnp.zeros((16,), jnp.float32)
        return 0
    lax.fori_loop(0, CH, zrow, 0)
    for t in range(STRIPE // CH):
        pltpu.sync_copy(sa_v, agg_sp.at[pl.ds(s * STRIPE + t * CH, CH), :])
    plsc.subcore_barrier()

    def gather(j, buf, sem):
        pltpu.async_copy(hwd_hbm.at[c].at[row_v.at[j]], buf, sem)

    def gather_wait(buf, sem):
        pltpu.make_async_copy(hwd_hbm.at[c].at[row_v.at[0]], buf, sem).wait()

    def scatter(j, buf, sem):
        pltpu.async_copy(buf, agg_sp.at[col_v.at[j]], sem, add=True)

    def scatter_wait(buf, sem):
        pltpu.make_async_copy(buf, agg_sp.at[col_v.at[0]], sem).wait()

    def scale(j, src, dst):
        def group(g, _):
            wv = w_v[j, pl.ds(g * 16, 16)]
            for k in range(16):
                w = wv[k]
                e = g * 16 + k
                for f in range(HD // 16):
                    dst[e, pl.ds(f * 16, 16)] = src[e, pl.ds(f * 16, 16)] * w
            return 0
        lax.fori_loop(0, CH // 16, group, 0)

    for half in range(2):
        pltpu.sync_copy(row_hbm.at[s, pl.ds(half * NH, NH)], row_v)
        pltpu.sync_copy(col_hbm.at[s, pl.ds(half * NH, NH)], col_v)
        pltpu.sync_copy(w_hbm.at[s, pl.ds(half * NH, NH)], w_v)
        gather(0, ga_v, gsa)
        gather(1, gb_v, gsb)

        def pipe(t, _):
            j0 = 2 * t
            j1 = j0 + 1
            j2 = j0 + 2
            j3 = j0 + 3
            gather_wait(ga_v, gsa)

            @pl.when(t > 0)
            def _():
                scatter_wait(sa_v, ssa)
            scale(j0, ga_v, sa_v)
            scatter(j0, sa_v, ssa)

            @pl.when(j2 < NH)
            def _():
                gather(j2, ga_v, gsa)
            gather_wait(gb_v, gsb)

            @pl.when(t > 0)
            def _():
                scatter_wait(sb_v, ssb)
            scale(j1, gb_v, sb_v)
            scatter(j1, sb_v, ssb)

            @pl.when(j3 < NH)
            def _():
                gather(j3, gb_v, gsb)
            return 0
        lax.fori_loop(0, NH // 2, pipe, 0)
        scatter_wait(sa_v, ssa)
        scatter_wait(sb_v, ssb)
    plsc.subcore_barrier()
    pltpu.sync_copy(agg_sp.at[pl.ds(s * STRIPE, STRIPE), :],
                    out_hbm.at[c, pl.ds(s * STRIPE, STRIPE), :])


def _row_spec(shape):
    nd = len(shape)
    blk = (NB,) + tuple(shape[1:])
    return pl.BlockSpec(blk, lambda i: (i,) + (0,) * (nd - 1))


def _full_spec(shape):
    nd = len(shape)
    return pl.BlockSpec(tuple(shape), lambda i: (0,) * nd)


def _split_spec():
    return pl.BlockSpec((NCORE, NB, HD), lambda i: (0, i, 0))


def _dinv_spec():
    return pl.BlockSpec((4, NB), lambda i: (0, i))


def _k1_body(x_ref, linw_ref, linb_ref, convw_ref, degp_ref, comb_ref,
             acc_ref, hwd_ref, dinv_ref):
    h = jnp.maximum(
        jnp.dot(x_ref[...], linw_ref[...],
                preferred_element_type=jnp.float32) + linb_ref[...], 0.0)
    deg = degp_ref[0, :, :, 0] + degp_ref[1, :, :, 0]
    dinv = jnp.where(deg > 0, lax.rsqrt(deg), 0.0)
    dinv_ref[...] = dinv
    acc_ref[...] = comb_ref[0, 0] * h
    hw = jnp.dot(h, convw_ref[...], preferred_element_type=jnp.float32)
    hwd = hw * dinv[0][:, None]
    hwd_ref[0] = hwd[:, :HD]
    hwd_ref[1] = hwd[:, HD:]


def _tc_k1(x_p, lin_w, lin_b, conv_w0, degp, comb):
    return pl.pallas_call(
        _k1_body,
        grid=(GRID,),
        in_specs=[
            _row_spec((NP, D)),
            _full_spec((D, D)),
            _full_spec((1, D)),
            _full_spec((D, D)),
            pl.BlockSpec((NCORE, 4, NB, 16), lambda i: (0, 0, i, 0)),
            _full_spec((1, 5)),
        ],
        out_specs=[_row_spec((NP, D)), _split_spec(), _dinv_spec()],
        out_shape=[
            jax.ShapeDtypeStruct((NP, D), jnp.float32),
            jax.ShapeDtypeStruct((NCORE, NP, HD), jnp.float32),
            jax.ShapeDtypeStruct((4, NP), jnp.float32),
        ],
    )(x_p, lin_w, lin_b, conv_w0, degp, comb)


def _step_body(a, agg_ref, dinv_ref, convb_ref, convw_ref, accin_ref,
               comb_ref, acc_ref, hwd_ref):
    aggf = jnp.concatenate([agg_ref[0], agg_ref[1]], axis=1)
    h = jnp.maximum(
        dinv_ref[a - 1][:, None] * aggf + convb_ref[...], 0.0)
    acc_ref[...] = accin_ref[...] + comb_ref[0, a] * h
    hw = jnp.dot(h, convw_ref[...], preferred_element_type=jnp.float32)
    hwd = hw * dinv_ref[a][:, None]
    hwd_ref[0] = hwd[:, :HD]
    hwd_ref[1] = hwd[:, HD:]


def _tc_step(a, agg, dinv, conv_b_prev, conv_w, acc, comb):
    return pl.pallas_call(
        functools.partial(_step_body, a),
        grid=(GRID,),
        in_specs=[
            _split_spec(),
            _dinv_spec(),
            _full_spec((1, D)),
            _full_spec((D, D)),
            _row_spec((NP, D)),
            _full_spec((1, 5)),
        ],
        out_specs=[_row_spec((NP, D)), _split_spec()],
        out_shape=[
            jax.ShapeDtypeStruct((NP, D), jnp.float32),
            jax.ShapeDtypeStruct((NCORE, NP, HD), jnp.float32),
        ],
    )(agg, dinv, conv_b_prev, conv_w, acc, comb)


def _final_body(agg_ref, dinv_ref, convb_ref, accin_ref, comb_ref, out_ref):
    aggf = jnp.concatenate([agg_ref[0], agg_ref[1]], axis=1)
    h = jnp.maximum(dinv_ref[3][:, None] * aggf + convb_ref[...], 0.0)
    out = accin_ref[...] + comb_ref[0, 4] * h
    m = jnp.max(out, axis=-1, keepdims=True)
    lse = jnp.log(jnp.sum(jnp.exp(out - m), axis=-1, keepdims=True)) + m
    out_ref[...] = out - lse


def _tc_final(agg, dinv, conv_b3, acc, comb):
    return pl.pallas_call(
        _final_body,
        grid=(GRID,),
        in_specs=[
            _split_spec(),
            _dinv_spec(),
            _full_spec((1, D)),
            _row_spec((NP, D)),
            _full_spec((1, 5)),
        ],
        out_specs=_row_spec((NP, D)),
        out_shape=jax.ShapeDtypeStruct((NP, D), jnp.float32),
    )(agg, dinv, conv_b3, acc, comb)


def kernel(x, edge_index, edge_attr, params):
    lin_w = params["lin_W"][-1]
    lin_b = params["lin_b"][-1].reshape(1, D)
    conv_w = params["conv_W"][-1]
    conv_b = params["conv_b"][-1]
    comb = params["comb_w"][-1].reshape(1, 5)

    row = edge_index[:, 0, :].astype(jnp.int32)
    col = edge_index[:, 1, :].astype(jnp.int32)
    pad = EP - E
    pad_idx = (jnp.arange(pad, dtype=jnp.int32) * 7) % N
    pad_idx4 = jnp.broadcast_to(pad_idx, (4, pad))
    shp = (4, NSUB, NCH, CH)
    row_t = jnp.concatenate([row, pad_idx4], axis=1).reshape(shp)
    col_t = jnp.concatenate([col, pad_idx4], axis=1).reshape(shp)
    w_t = jnp.concatenate(
        [edge_attr.astype(jnp.float32), jnp.zeros((4, pad), jnp.float32)],
        axis=1).reshape(shp)

    x_p = jnp.concatenate([x, jnp.zeros((NP - N, D), jnp.float32)], axis=0)

    degp = _deg_kernel(col_t, w_t)
    acc, hwd, dinv = _tc_k1(x_p, lin_w, lin_b, conv_w[0], degp, comb)

    for a in range(1, 4):
        agg = _conv_kernel(hwd, row_t[a - 1], col_t[a - 1], w_t[a - 1])
        acc, hwd = _tc_step(a, agg, dinv, conv_b[a - 1].reshape(1, D),
                            conv_w[a], acc, comb)
    agg = _conv_kernel(hwd, row_t[3], col_t[3], w_t[3])
    out = _tc_final(agg, dinv, conv_b[3].reshape(1, D), acc, comb)
    return out[:N]


# speedup vs baseline: 29.5554x; 1.0516x over previous
import functools

import jax
import jax.numpy as jnp
from jax import lax
from jax.experimental import pallas as pl
from jax.experimental.pallas import tpu as pltpu
from jax.experimental.pallas import tpu_sc as plsc

N = 10000
NP = 10240
D = 128
HD = 64
E = 320000
NSUB = 16
NCORE = 2
CH = 128
NCH = 160
NH = NCH // 2
EP = NSUB * NCH * CH
NB = 2048
GRID = NP // NB
STRIPE = NP // NSUB

_SC_MESH = plsc.VectorSubcoreMesh(core_axis_name="c", subcore_axis_name="s")
_SC_PARAMS = pltpu.CompilerParams(use_tc_tiling_on_sc=False)


@functools.partial(
    pl.kernel,
    out_type=jax.ShapeDtypeStruct((NCORE, 4, NP, 16), jnp.float32),
    mesh=_SC_MESH,
    compiler_params=_SC_PARAMS,
    scratch_types=[
        pltpu.VMEM((NCH, CH), jnp.int32),
        pltpu.VMEM((NCH, CH), jnp.float32),
        pltpu.VMEM((CH, 16), jnp.float32),
        pltpu.VMEM((CH, 16), jnp.float32),
        pltpu.VMEM((CH, 16), jnp.float32),
        pltpu.VMEM_SHARED((NP, 16), jnp.float32),
        pltpu.SemaphoreType.DMA,
        pltpu.SemaphoreType.DMA,
    ],
)
def _deg_kernel(col_hbm, w_hbm, out_hbm, col_v, w_v, wrow_a, wrow_b, zbuf,
                deg_sp, ssa, ssb):
    c = lax.axis_index("c")
    s = lax.axis_index("s")

    def zrow(i, _):
        zbuf[i, pl.ds(0, 16)] = jnp.zeros((16,), jnp.float32)
        return 0
    lax.fori_loop(0, CH, zrow, 0)

    def build(j, buf):
        def group(g, _):
            wv = w_v[j, pl.ds(g * 16, 16)]
            for k in range(16):
                buf[g * 16 + k, pl.ds(0, 16)] = jnp.full(
                    (16,), wv[k], jnp.float32)
            return 0
        lax.fori_loop(0, CH // 16, group, 0)

    def scatter(j, buf, sem):
        pltpu.async_copy(buf, deg_sp.at[col_v.at[j]], sem, add=True)

    def scatter_wait(buf, sem):
        pltpu.make_async_copy(buf, deg_sp.at[col_v.at[0]], sem).wait()

    for a in range(4):
        for t in range(STRIPE // CH):
            pltpu.sync_copy(
                zbuf, deg_sp.at[pl.ds(s * STRIPE + t * CH, CH), :])
        plsc.subcore_barrier()
        pltpu.sync_copy(col_hbm.at[a, s], col_v)
        pltpu.sync_copy(w_hbm.at[a, s], w_v)

        def chunk(t, _):
            ja = 4 * t + c
            jb = ja + 2
            build(ja, wrow_a)

            @pl.when(t > 0)
            def _():
                scatter_wait(wrow_b, ssb)
            scatter(ja, wrow_a, ssa)
            build(jb, wrow_b)
            scatter_wait(wrow_a, ssa)
            scatter(jb, wrow_b, ssb)
            return 0
        lax.fori_loop(0, NCH // 4, chunk, 0)
        scatter_wait(wrow_b, ssb)
        plsc.subcore_barrier()
        pltpu.sync_copy(deg_sp.at[pl.ds(s * STRIPE, STRIPE), :],
                        out_hbm.at[c, a, pl.ds(s * STRIPE, STRIPE), :])
        plsc.subcore_barrier()


@functools.partial(
    pl.kernel,
    out_type=jax.ShapeDtypeStruct((NCORE, NP, HD), jnp.float32),
    mesh=_SC_MESH,
    compiler_params=_SC_PARAMS,
    scratch_types=[
        pltpu.VMEM((NH, CH), jnp.int32),
        pltpu.VMEM((NH, CH), jnp.int32),
        pltpu.VMEM((NH, CH), jnp.float32),
        pltpu.VMEM((CH, HD), jnp.float32),
        pltpu.VMEM((CH, HD), jnp.float32),
        pltpu.VMEM((CH, HD), jnp.float32),
        pltpu.VMEM((CH, HD), jnp.float32),
        pltpu.VMEM_SHARED((NP, HD), jnp.float32),
        pltpu.SemaphoreType.DMA,
        pltpu.SemaphoreType.DMA,
        pltpu.SemaphoreType.DMA,
        pltpu.SemaphoreType.DMA,
    ],
)
def _conv_kernel(hwd_hbm, row_hbm, col_hbm, w_hbm, out_hbm,
                 row_v, col_v, w_v, ga_v, gb_v, sa_v, sb_v, agg_sp,
                 gsa, gsb, ssa, ssb):
    c = lax.axis_index("c")
    s = lax.axis_index("s")
    def zrow(i, _):
        for f in range(HD // 16):
            sa_v[i, pl.ds(f * 16, 16)] = jnp.zeros((16,), jnp.float32)
        return 0
    lax.fori_loop(0, CH, zrow, 0)
    for t in range(STRIPE // CH):
        pltpu.sync_copy(sa_v, agg_sp.at[pl.ds(s * STRIPE + t * CH, CH), :])
    plsc.subcore_barrier()

    def gather(j, buf, sem):
        pltpu.async_copy(hwd_hbm.at[c].at[row_v.at[j]], buf, sem)

    def gather_wait(buf, sem):
        pltpu.make_async_copy(hwd_hbm.at[c].at[row_v.at[0]], buf, sem).wait()

    def scatter(j, buf, sem):
        pltpu.async_copy(buf, agg_sp.at[col_v.at[j]], sem, add=True)

    def scatter_wait(buf, sem):
        pltpu.make_async_copy(buf, agg_sp.at[col_v.at[0]], sem).wait()

    def scale(j, src, dst):
        def group(g, _):
            wv = w_v[j, pl.ds(g * 16, 16)]
            for k in range(16):
                w = wv[k]
                e = g * 16 + k
                for f in range(HD // 16):
                    dst[e, pl.ds(f * 16, 16)] = src[e, pl.ds(f * 16, 16)] * w
            return 0
        lax.fori_loop(0, CH // 16, group, 0)

    for half in range(2):
        pltpu.sync_copy(row_hbm.at[s, pl.ds(half * NH, NH)], row_v)
        pltpu.sync_copy(col_hbm.at[s, pl.ds(half * NH, NH)], col_v)
        pltpu.sync_copy(w_hbm.at[s, pl.ds(half * NH, NH)], w_v)
        gather(0, ga_v, gsa)
        gather(1, gb_v, gsb)

        def pipe(t, _):
            j0 = 2 * t
            j1 = j0 + 1
            j2 = j0 + 2
            j3 = j0 + 3
            gather_wait(ga_v, gsa)
            scale(j0, ga_v, sa_v)

            @pl.when(t > 0)
            def _():
                scatter_wait(sb_v, ssb)
            scatter(j0, sa_v, ssa)

            @pl.when(j2 < NH)
            def _():
                gather(j2, ga_v, gsa)
            gather_wait(gb_v, gsb)
            scale(j1, gb_v, sb_v)
            scatter_wait(sa_v, ssa)
            scatter(j1, sb_v, ssb)

            @pl.when(j3 < NH)
            def _():
                gather(j3, gb_v, gsb)
            return 0
        lax.fori_loop(0, NH // 2, pipe, 0)
        scatter_wait(sb_v, ssb)
    plsc.subcore_barrier()
    pltpu.sync_copy(agg_sp.at[pl.ds(s * STRIPE, STRIPE), :],
                    out_hbm.at[c, pl.ds(s * STRIPE, STRIPE), :])


def _row_spec(shape):
    nd = len(shape)
    blk = (NB,) + tuple(shape[1:])
    return pl.BlockSpec(blk, lambda i: (i,) + (0,) * (nd - 1))


def _full_spec(shape):
    nd = len(shape)
    return pl.BlockSpec(tuple(shape), lambda i: (0,) * nd)


def _split_spec():
    return pl.BlockSpec((NCORE, NB, HD), lambda i: (0, i, 0))


def _dinv_spec():
    return pl.BlockSpec((4, NB), lambda i: (0, i))


def _k1_body(x_ref, linw_ref, linb_ref, convw_ref, degp_ref, comb_ref,
             acc_ref, hwd_ref, dinv_ref):
    h = jnp.maximum(
        jnp.dot(x_ref[...], linw_ref[...],
                preferred_element_type=jnp.float32) + linb_ref[...], 0.0)
    deg = degp_ref[0, :, :, 0] + degp_ref[1, :, :, 0]
    dinv = jnp.where(deg > 0, lax.rsqrt(deg), 0.0)
    dinv_ref[...] = dinv
    acc_ref[...] = comb_ref[0, 0] * h
    hw = jnp.dot(h, convw_ref[...], preferred_element_type=jnp.float32)
    hwd = hw * dinv[0][:, None]
    hwd_ref[0] = hwd[:, :HD]
    hwd_ref[1] = hwd[:, HD:]


def _tc_k1(x_p, lin_w, lin_b, conv_w0, degp, comb):
    return pl.pallas_call(
        _k1_body,
        grid=(GRID,),
        in_specs=[
            _row_spec((NP, D)),
            _full_spec((D, D)),
            _full_spec((1, D)),
            _full_spec((D, D)),
            pl.BlockSpec((NCORE, 4, NB, 16), lambda i: (0, 0, i, 0)),
            _full_spec((1, 5)),
        ],
        out_specs=[_row_spec((NP, D)), _split_spec(), _dinv_spec()],
        out_shape=[
            jax.ShapeDtypeStruct((NP, D), jnp.float32),
            jax.ShapeDtypeStruct((NCORE, NP, HD), jnp.float32),
            jax.ShapeDtypeStruct((4, NP), jnp.float32),
        ],
    )(x_p, lin_w, lin_b, conv_w0, degp, comb)


def _step_body(a, agg_ref, dinv_ref, convb_ref, convw_ref, accin_ref,
               comb_ref, acc_ref, hwd_ref):
    aggf = jnp.concatenate([agg_ref[0], agg_ref[1]], axis=1)
    h = jnp.maximum(
        dinv_ref[a - 1][:, None] * aggf + convb_ref[...], 0.0)
    acc_ref[...] = accin_ref[...] + comb_ref[0, a] * h
    hw = jnp.dot(h, convw_ref[...], preferred_element_type=jnp.float32)
    hwd = hw * dinv_ref[a][:, None]
    hwd_ref[0] = hwd[:, :HD]
    hwd_ref[1] = hwd[:, HD:]


def _tc_step(a, agg, dinv, conv_b_prev, conv_w, acc, comb):
    return pl.pallas_call(
        functools.partial(_step_body, a),
        grid=(GRID,),
        in_specs=[
            _split_spec(),
            _dinv_spec(),
            _full_spec((1, D)),
            _full_spec((D, D)),
            _row_spec((NP, D)),
            _full_spec((1, 5)),
        ],
        out_specs=[_row_spec((NP, D)), _split_spec()],
        out_shape=[
            jax.ShapeDtypeStruct((NP, D), jnp.float32),
            jax.ShapeDtypeStruct((NCORE, NP, HD), jnp.float32),
        ],
    )(agg, dinv, conv_b_prev, conv_w, acc, comb)


def _final_body(agg_ref, dinv_ref, convb_ref, accin_ref, comb_ref, out_ref):
    aggf = jnp.concatenate([agg_ref[0], agg_ref[1]], axis=1)
    h = jnp.maximum(dinv_ref[3][:, None] * aggf + convb_ref[...], 0.0)
    out = accin_ref[...] + comb_ref[0, 4] * h
    m = jnp.max(out, axis=-1, keepdims=True)
    lse = jnp.log(jnp.sum(jnp.exp(out - m), axis=-1, keepdims=True)) + m
    out_ref[...] = out - lse


def _tc_final(agg, dinv, conv_b3, acc, comb):
    return pl.pallas_call(
        _final_body,
        grid=(GRID,),
        in_specs=[
            _split_spec(),
            _dinv_spec(),
            _full_spec((1, D)),
            _row_spec((NP, D)),
            _full_spec((1, 5)),
        ],
        out_specs=_row_spec((NP, D)),
        out_shape=jax.ShapeDtypeStruct((NP, D), jnp.float32),
    )(agg, dinv, conv_b3, acc, comb)


def kernel(x, edge_index, edge_attr, params):
    lin_w = params["lin_W"][-1]
    lin_b = params["lin_b"][-1].reshape(1, D)
    conv_w = params["conv_W"][-1]
    conv_b = params["conv_b"][-1]
    comb = params["comb_w"][-1].reshape(1, 5)

    row = edge_index[:, 0, :].astype(jnp.int32)
    col = edge_index[:, 1, :].astype(jnp.int32)
    pad = EP - E
    pad_idx = (jnp.arange(pad, dtype=jnp.int32) * 7) % N
    pad_idx4 = jnp.broadcast_to(pad_idx, (4, pad))
    shp = (4, NSUB, NCH, CH)
    row_t = jnp.concatenate([row, pad_idx4], axis=1).reshape(shp)
    col_t = jnp.concatenate([col, pad_idx4], axis=1).reshape(shp)
    w_t = jnp.concatenate(
        [edge_attr.astype(jnp.float32), jnp.zeros((4, pad), jnp.float32)],
        axis=1).reshape(shp)

    x_p = jnp.concatenate([x, jnp.zeros((NP - N, D), jnp.float32)], axis=0)

    degp = _deg_kernel(col_t, w_t)
    acc, hwd, dinv = _tc_k1(x_p, lin_w, lin_b, conv_w[0], degp, comb)

    for a in range(1, 4):
        agg = _conv_kernel(hwd, row_t[a - 1], col_t[a - 1], w_t[a - 1])
        acc, hwd = _tc_step(a, agg, dinv, conv_b[a - 1].reshape(1, D),
                            conv_w[a], acc, comb)
    agg = _conv_kernel(hwd, row_t[3], col_t[3], w_t[3])
    out = _tc_final(agg, dinv, conv_b[3].reshape(1, D), acc, comb)
    return out[:N]
